# Initial kernel scaffold; baseline (speedup 1.0000x reference)
import jax
import jax.numpy as jnp
from jax.experimental import pallas as pl


def kernel(user_emb, item_emb, graph_indices, graph_values, ii_indices, ii_values):
    raise NotImplementedError("write your pallas kernel here")


# speedup vs baseline: 2.5822x; 2.5822x over previous
import dataclasses
import functools

import jax
import jax.numpy as jnp
from jax import lax
from jax.experimental import pallas as pl
from jax.experimental.pallas import tpu as pltpu
from jax.experimental.pallas import tpu_sc as plsc

NU = 25000
NI = 25000
D = 64
LANES = 16
NTILES = 16
NSC = 2
CHUNK = 128
ZB = 32


def _ceil_to(x, m):
    return (x + m - 1) // m * m


_SPLAT_DNUMS = lax.GatherDimensionNumbers(
    offset_dims=(), collapsed_slice_dims=(0,), start_index_map=(0,))


def _splat(v16, j):
    idx = jnp.full((LANES, 1), j, jnp.int32)
    return lax.gather(v16, idx, _SPLAT_DNUMS, slice_sizes=(1,),
                      mode=lax.GatherScatterMode.PROMISE_IN_BOUNDS)


def _make_spmm(n_x_rows, n_out_rows, e_pad):
    half = n_out_rows // 2
    rows_per_tile = _ceil_to(-(-half // NTILES), ZB)
    acc_rows = rows_per_tile * NTILES
    dummy = half + LANES
    assert dummy < acc_rows
    e_per_tile = e_pad // NTILES
    assert e_per_tile % CHUNK == 0

    cp = pltpu.CompilerParams()
    if "needs_layout_passes" in pltpu.CompilerParams.__dataclass_fields__:
        cp = dataclasses.replace(cp, needs_layout_passes=False)
    if "use_tc_tiling_on_sc" in pltpu.CompilerParams.__dataclass_fields__:
        cp = dataclasses.replace(cp, use_tc_tiling_on_sc=False)

    @functools.partial(
        pl.kernel,
        out_type=jax.ShapeDtypeStruct((NSC * acc_rows, D), jnp.float32),
        mesh=plsc.VectorSubcoreMesh(core_axis_name="c", subcore_axis_name="s"),
        compiler_params=cp,
        scratch_types=[
            pltpu.VMEM((CHUNK,), jnp.int32),
            pltpu.VMEM((CHUNK,), jnp.int32),
            pltpu.VMEM((CHUNK,), jnp.int32),
            pltpu.VMEM((CHUNK,), jnp.float32),
            pltpu.VMEM((CHUNK, D), jnp.float32),
            pltpu.VMEM((ZB, D), jnp.float32),
            pltpu.VMEM_SHARED((acc_rows, D), jnp.float32),
        ],
    )
    def spmm_k(x_hbm, rows_hbm, cols_hbm, vals_hbm, out_hbm,
               rows_v, cols_v, idx_v, vals_v, gath_v, zero_v, acc):
        c = lax.axis_index("c")
        s = lax.axis_index("s")
        row_base = c * half

        zvec = jnp.zeros((LANES,), jnp.float32)
        for r in range(ZB):
            for dd in range(D // LANES):
                zero_v[r, pl.ds(dd * LANES, LANES)] = zvec
        t0 = s * rows_per_tile

        @pl.loop(0, rows_per_tile, step=ZB)
        def _(off):
            pltpu.sync_copy(zero_v, acc.at[pl.ds(t0 + off, ZB)])

        plsc.subcore_barrier()

        e0 = s * e_per_tile

        @pl.loop(0, e_per_tile, step=CHUNK)
        def _(eoff):
            eb = e0 + eoff
            pltpu.sync_copy(rows_hbm.at[pl.ds(eb, CHUNK)], rows_v)
            pltpu.sync_copy(cols_hbm.at[pl.ds(eb, CHUNK)], cols_v)
            pltpu.sync_copy(vals_hbm.at[pl.ds(eb, CHUNK)], vals_v)
            pltpu.sync_copy(x_hbm.at[cols_v], gath_v)
            for g in range(CHUNK // LANES):
                sl = pl.ds(g * LANES, LANES)
                loc = rows_v[sl] - row_base
                ok = (loc >= 0) & (loc < half)
                idx_v[sl] = jnp.where(ok, loc, dummy)
            for g in range(CHUNK // LANES):
                v16 = vals_v[pl.ds(g * LANES, LANES)]
                for j in range(LANES):
                    e = g * LANES + j
                    ve = _splat(v16, j)
                    for dd in range(D // LANES):
                        sl = pl.ds(dd * LANES, LANES)
                        gath_v[e, sl] = gath_v[e, sl] * ve
            pltpu.sync_copy(gath_v, acc.at[idx_v], add=True)

        plsc.subcore_barrier()

        out_base = c * acc_rows + s * rows_per_tile
        pltpu.sync_copy(acc.at[pl.ds(t0, rows_per_tile)],
                        out_hbm.at[pl.ds(out_base, rows_per_tile)])

    def spmm(x, rows, cols, vals):
        out = spmm_k(x, rows, cols, vals)
        return jnp.concatenate(
            [out[:half], out[acc_rows:acc_rows + half]], axis=0)

    return spmm


_G_E_PAD = _ceil_to(800000, NTILES * CHUNK)
_II_E_PAD = _ceil_to(400000, NTILES * CHUNK)

_spmm_graph = _make_spmm(NU + NI, NU + NI, _G_E_PAD)
_spmm_ii = _make_spmm(NI, NI, _II_E_PAD)


def _pad_edges(indices, values, e_pad, n_rows):
    e = values.shape[0]
    pad = e_pad - e
    rows = jnp.concatenate([indices[0], jnp.full((pad,), n_rows, jnp.int32)])
    cols = jnp.concatenate([indices[1], jnp.zeros((pad,), jnp.int32)])
    vals = jnp.concatenate([values, jnp.zeros((pad,), jnp.float32)])
    return rows, cols, vals


def kernel(user_emb, item_emb, graph_indices, graph_values, ii_indices, ii_values):
    ii_rows, ii_cols, ii_vals = _pad_edges(ii_indices, ii_values, _II_E_PAD, NI)
    cur = item_emb
    acc = item_emb
    for _ in range(2):
        cur = _spmm_ii(cur, ii_rows, ii_cols, ii_vals)
        acc = acc + cur
    items_emb = acc * (1.0 / 3.0)

    g_rows, g_cols, g_vals = _pad_edges(graph_indices, graph_values, _G_E_PAD,
                                        NU + NI)
    all_emb = jnp.concatenate([user_emb, items_emb], axis=0)
    s = all_emb
    for _ in range(3):
        all_emb = _spmm_graph(all_emb, g_rows, g_cols, g_vals)
        s = s + all_emb
    light = s * 0.25
    return light[:NU], light[NU:]


# speedup vs baseline: 4.6156x; 1.7875x over previous
import dataclasses
import functools

import jax
import jax.numpy as jnp
from jax import lax
from jax.experimental import pallas as pl
from jax.experimental.pallas import tpu as pltpu
from jax.experimental.pallas import tpu_sc as plsc

NU = 25000
NI = 25000
D = 64
LANES = 16
NTILES = 16
NSC = 2
CHUNK = 128
REC = 3 * CHUNK
ZB = 32


def _ceil_to(x, m):
    return (x + m - 1) // m * m


_SPLAT_DNUMS = lax.GatherDimensionNumbers(
    offset_dims=(), collapsed_slice_dims=(0,), start_index_map=(0,))


def _splat(v16, j):
    idx = jnp.full((LANES, 1), j, jnp.int32)
    return lax.gather(v16, idx, _SPLAT_DNUMS, slice_sizes=(1,),
                      mode=lax.GatherScatterMode.PROMISE_IN_BOUNDS)


def _make_spmm(n_x_rows, n_out_rows, e_pad):
    half = n_out_rows // 2
    rows_per_tile = _ceil_to(-(-half // NTILES), ZB)
    acc_rows = rows_per_tile * NTILES
    dummy = half + LANES
    assert dummy < acc_rows
    e_per_tile = e_pad // NTILES
    nc = e_per_tile // CHUNK
    assert e_per_tile % CHUNK == 0 and nc % 2 == 0
    nchunks_total = e_pad // CHUNK

    cp = pltpu.CompilerParams()
    if "needs_layout_passes" in pltpu.CompilerParams.__dataclass_fields__:
        cp = dataclasses.replace(cp, needs_layout_passes=False)
    if "use_tc_tiling_on_sc" in pltpu.CompilerParams.__dataclass_fields__:
        cp = dataclasses.replace(cp, use_tc_tiling_on_sc=False)

    @functools.partial(
        pl.kernel,
        out_type=jax.ShapeDtypeStruct((NSC * acc_rows, D), jnp.float32),
        mesh=plsc.VectorSubcoreMesh(core_axis_name="c", subcore_axis_name="s"),
        compiler_params=cp,
        scratch_types=[
            pltpu.VMEM((2, REC), jnp.int32),
            pltpu.VMEM((2, CHUNK), jnp.int32),
            pltpu.VMEM((2, CHUNK, D), jnp.float32),
            pltpu.VMEM((ZB, D), jnp.float32),
            pltpu.VMEM_SHARED((acc_rows, D), jnp.float32),
            pltpu.SemaphoreType.DMA,
            pltpu.SemaphoreType.DMA,
            pltpu.SemaphoreType.DMA,
            pltpu.SemaphoreType.DMA,
            pltpu.SemaphoreType.DMA,
            pltpu.SemaphoreType.DMA,
        ],
    )
    def spmm_k(x_hbm, ebuf_hbm, out_hbm,
               ebuf_v, idx_v, gath_v, zero_v, acc,
               se0, se1, sg0, sg1, ss0, ss1):
        sem_e = (se0, se1)
        sem_g = (sg0, sg1)
        sem_s = (ss0, ss1)
        c = lax.axis_index("c")
        s = lax.axis_index("s")
        row_base = c * half

        zvec = jnp.zeros((LANES,), jnp.float32)
        for r in range(ZB):
            for dd in range(D // LANES):
                zero_v[r, pl.ds(dd * LANES, LANES)] = zvec
        t0 = s * rows_per_tile

        @pl.loop(0, rows_per_tile, step=ZB)
        def _(off):
            pltpu.sync_copy(zero_v, acc.at[pl.ds(t0 + off, ZB)])

        plsc.subcore_barrier()

        c0 = s * nc

        def issue_idx(k, b):
            pltpu.async_copy(ebuf_hbm.at[c0 + k], ebuf_v.at[b], sem_e[b])

        def wait_idx(b):
            pltpu.make_async_copy(
                ebuf_hbm.at[0], ebuf_v.at[b], sem_e[b]).wait()

        def issue_gather(b):
            pltpu.async_copy(
                x_hbm.at[ebuf_v.at[b, pl.ds(CHUNK, CHUNK)]],
                gath_v.at[b], sem_g[b])

        def wait_gather(b):
            pltpu.make_async_copy(
                x_hbm.at[pl.ds(0, CHUNK)], gath_v.at[b], sem_g[b]).wait()

        def issue_scatter(b):
            pltpu.async_copy(gath_v.at[b], acc.at[idx_v.at[b]], sem_s[b],
                             add=True)

        def wait_scatter(b):
            pltpu.make_async_copy(
                gath_v.at[b], acc.at[pl.ds(0, CHUNK)], sem_s[b]).wait()

        def compute(b):
            for g in range(CHUNK // LANES):
                sl = pl.ds(g * LANES, LANES)
                loc = ebuf_v[b, sl] - row_base
                ok = (loc >= 0) & (loc < half)
                idx_v[b, sl] = jnp.where(ok, loc, dummy)
            for g in range(CHUNK // LANES):
                v16 = plsc.bitcast(
                    ebuf_v[b, pl.ds(2 * CHUNK + g * LANES, LANES)],
                    jnp.float32)
                for j in range(LANES):
                    e = g * LANES + j
                    ve = _splat(v16, j)
                    for dd in range(D // LANES):
                        sl = pl.ds(dd * LANES, LANES)
                        gath_v[b, e, sl] = gath_v[b, e, sl] * ve

        def body(k, b):
            wait_gather(b)
            compute(b)

            @pl.when(k + 1 < nc)
            def _():
                wait_idx(1 - b)

                @pl.when(k >= 1)
                def _():
                    wait_scatter(1 - b)

                issue_gather(1 - b)

            @pl.when(k + 2 < nc)
            def _():
                issue_idx(k + 2, b)

            issue_scatter(b)

        issue_idx(0, 0)
        wait_idx(0)
        issue_gather(0)
        issue_idx(1, 1)

        @pl.loop(0, nc, step=2)
        def _(k):
            body(k, 0)
            body(k + 1, 1)

        wait_scatter(1)
        plsc.subcore_barrier()

        out_base = c * acc_rows + s * rows_per_tile
        pltpu.sync_copy(acc.at[pl.ds(t0, rows_per_tile)],
                        out_hbm.at[pl.ds(out_base, rows_per_tile)])

    def spmm(x, ebuf):
        out = spmm_k(x, ebuf)
        return jnp.concatenate(
            [out[:half], out[acc_rows:acc_rows + half]], axis=0)

    return spmm


_G_E_PAD = _ceil_to(800000, 2 * NTILES * CHUNK)
_II_E_PAD = _ceil_to(400000, 2 * NTILES * CHUNK)

_spmm_graph = _make_spmm(NU + NI, NU + NI, _G_E_PAD)
_spmm_ii = _make_spmm(NI, NI, _II_E_PAD)


def _pack_edges(indices, values, e_pad, n_rows):
    e = values.shape[0]
    pad = e_pad - e
    rows = jnp.concatenate([indices[0], jnp.full((pad,), n_rows, jnp.int32)])
    cols = jnp.concatenate([indices[1], jnp.zeros((pad,), jnp.int32)])
    vals = jnp.concatenate([values, jnp.zeros((pad,), jnp.float32)])
    nch = e_pad // CHUNK
    rec = jnp.concatenate([rows.reshape(nch, CHUNK),
                           cols.reshape(nch, CHUNK),
                           lax.bitcast_convert_type(vals, jnp.int32)
                              .reshape(nch, CHUNK)], axis=1)
    return rec


def kernel(user_emb, item_emb, graph_indices, graph_values, ii_indices, ii_values):
    ii_rec = _pack_edges(ii_indices, ii_values, _II_E_PAD, NI)
    cur = item_emb
    acc = item_emb
    for _ in range(2):
        cur = _spmm_ii(cur, ii_rec)
        acc = acc + cur
    items_emb = acc * (1.0 / 3.0)

    g_rec = _pack_edges(graph_indices, graph_values, _G_E_PAD, NU + NI)
    all_emb = jnp.concatenate([user_emb, items_emb], axis=0)
    s = all_emb
    for _ in range(3):
        all_emb = _spmm_graph(all_emb, g_rec)
        s = s + all_emb
    light = s * 0.25
    return light[:NU], light[NU:]


# speedup vs baseline: 6.7777x; 1.4684x over previous
import dataclasses
import functools

import jax
import jax.numpy as jnp
from jax import lax
from jax.experimental import pallas as pl
from jax.experimental.pallas import tpu as pltpu
from jax.experimental.pallas import tpu_sc as plsc

NU = 25000
NI = 25000
D = 64
LANES = 16
NTILES = 16
NSC = 2
CHUNK = 128
REC = 3 * CHUNK
ZB = 32
BLK = 1024
STG = 1536


def _ceil_to(x, m):
    return (x + m - 1) // m * m


_SPLAT_DNUMS = lax.GatherDimensionNumbers(
    offset_dims=(), collapsed_slice_dims=(0,), start_index_map=(0,))


def _splat(v16, j):
    idx = jnp.full((LANES, 1), j, jnp.int32)
    return lax.gather(v16, idx, _SPLAT_DNUMS, slice_sizes=(1,),
                      mode=lax.GatherScatterMode.PROMISE_IN_BOUNDS)


def _compiler_params():
    cp = pltpu.CompilerParams()
    if "needs_layout_passes" in pltpu.CompilerParams.__dataclass_fields__:
        cp = dataclasses.replace(cp, needs_layout_passes=False)
    if "use_tc_tiling_on_sc" in pltpu.CompilerParams.__dataclass_fields__:
        cp = dataclasses.replace(cp, use_tc_tiling_on_sc=False)
    return cp


_MESH = dict(core_axis_name="c", subcore_axis_name="s")


def _make_partition(n_out_rows, e_pad):
    half = n_out_rows // 2
    rows_per_tile = _ceil_to(-(-half // NTILES), ZB)
    acc_rows = rows_per_tile * NTILES
    dummy = half + LANES
    e_per_tile = e_pad // NTILES
    nc = e_per_tile // CHUNK
    cap = _ceil_to(nc * CHUNK, BLK) + BLK

    @functools.partial(
        pl.kernel,
        out_type=[
            jax.ShapeDtypeStruct((NSC, NTILES * cap), jnp.int32),
            jax.ShapeDtypeStruct((NSC, NTILES * cap), jnp.int32),
            jax.ShapeDtypeStruct((NSC, NTILES * cap), jnp.int32),
            jax.ShapeDtypeStruct((NSC, NTILES, LANES), jnp.int32),
        ],
        mesh=plsc.VectorSubcoreMesh(**_MESH),
        compiler_params=_compiler_params(),
        scratch_types=[
            pltpu.VMEM((2, REC), jnp.int32),
            pltpu.VMEM((STG,), jnp.int32),
            pltpu.VMEM((STG,), jnp.int32),
            pltpu.VMEM((STG,), jnp.int32),
            pltpu.VMEM((LANES,), jnp.int32),
            pltpu.SemaphoreType.DMA,
            pltpu.SemaphoreType.DMA,
        ],
    )
    def part_k(ebuf_hbm, orow_hbm, ocol_hbm, oval_hbm, ocnt_hbm,
               ebuf_v, srow_v, scol_v, sval_v, cnt_v, se0, se1):
        sem_e = (se0, se1)
        c = lax.axis_index("c")
        s = lax.axis_index("s")
        row_base = c * half
        c0 = s * nc
        obase = s * cap

        def issue_idx(k, b):
            pltpu.async_copy(ebuf_hbm.at[c0 + k], ebuf_v.at[b], sem_e[b])

        def wait_idx(b):
            pltpu.make_async_copy(
                ebuf_hbm.at[0], ebuf_v.at[b], sem_e[b]).wait()

        def flush(optr):
            off = pl.multiple_of(obase + optr, BLK)
            pltpu.sync_copy(srow_v.at[pl.ds(0, BLK)],
                            orow_hbm.at[c, pl.ds(off, BLK)])
            pltpu.sync_copy(scol_v.at[pl.ds(0, BLK)],
                            ocol_hbm.at[c, pl.ds(off, BLK)])
            pltpu.sync_copy(sval_v.at[pl.ds(0, BLK)],
                            oval_hbm.at[c, pl.ds(off, BLK)])

        def shift():
            for g in range(CHUNK // LANES):
                sl_src = pl.ds(BLK + g * LANES, LANES)
                sl_dst = pl.ds(g * LANES, LANES)
                srow_v[sl_dst] = srow_v[sl_src]
                scol_v[sl_dst] = scol_v[sl_src]
                sval_v[sl_dst] = sval_v[sl_src]

        issue_idx(0, 0)
        issue_idx(1, 1)

        def chunk_body(k, carry):
            cur, optr = carry
            b0 = k % 2
            def with_buf(b):
                wait_idx(b)

                @pl.when(k + 2 < nc)
                def _():
                    issue_idx(k + 2, b)

                cur2 = cur
                for g in range(CHUNK // LANES):
                    sl = pl.ds(g * LANES, LANES)
                    rows16 = ebuf_v[b, sl]
                    loc = rows16 - row_base
                    ok = (loc >= 0) & (loc < half)
                    n16 = plsc.all_reduce_population_count(ok)
                    cnt = jnp.max(n16, axis=0) if n16.ndim else n16
                    plsc.store_compressed(
                        srow_v.at[pl.ds(cur2, LANES)], loc, mask=ok)
                    plsc.store_compressed(
                        scol_v.at[pl.ds(cur2, LANES)],
                        ebuf_v[b, pl.ds(CHUNK + g * LANES, LANES)], mask=ok)
                    plsc.store_compressed(
                        sval_v.at[pl.ds(cur2, LANES)],
                        ebuf_v[b, pl.ds(2 * CHUNK + g * LANES, LANES)], mask=ok)
                    cur2 = cur2 + cnt
                return cur2

            cur = lax.cond(b0 == 0, lambda: with_buf(0), lambda: with_buf(1))

            def do_flush():
                flush(optr)
                shift()
                return cur - BLK, optr + BLK

            cur, optr = lax.cond(cur >= BLK, do_flush, lambda: (cur, optr))
            return cur, optr

        cur, optr = lax.fori_loop(0, nc, chunk_body, (jnp.int32(0),
                                                      jnp.int32(0)))

        zero16 = jnp.zeros((LANES,), jnp.int32)
        dummy16 = jnp.full((LANES,), dummy, jnp.int32)
        pad_to = _pad_target(cur)
        base = cur

        def pad_body(g, _):
            off = base + g * LANES

            @pl.when(off < pad_to)
            def _():
                srow_v[pl.ds(off, LANES)] = dummy16
                scol_v[pl.ds(off, LANES)] = zero16
                sval_v[pl.ds(off, LANES)] = zero16
            return 0

        lax.fori_loop(0, (STG - BLK) // LANES, pad_body, 0)
        cur = pad_to

        nrec = (optr + cur) // CHUNK

        def do_flush2():
            flush(optr)
            shift()
            return cur - BLK, optr + BLK

        cur, optr = lax.cond(cur >= BLK, do_flush2, lambda: (cur, optr))
        flush(optr)

        cnt_v[pl.ds(0, LANES)] = jnp.full((LANES,), 1, jnp.int32) * nrec
        pltpu.sync_copy(cnt_v, ocnt_hbm.at[c, s])

    return part_k, cap, acc_rows, rows_per_tile, dummy, half


def _pad_target(cur):
    rec = (cur + CHUNK - 1) // CHUNK
    rec = rec + (rec % 2)
    rec = jnp.maximum(rec, 2)
    return rec * CHUNK


def _make_spmm(n_x_rows, n_out_rows, e_pad):
    part_k, cap, acc_rows, rows_per_tile, dummy, half = _make_partition(
        n_out_rows, e_pad)

    @functools.partial(
        pl.kernel,
        out_type=jax.ShapeDtypeStruct((NSC * acc_rows, D), jnp.float32),
        mesh=plsc.VectorSubcoreMesh(**_MESH),
        compiler_params=_compiler_params(),
        scratch_types=[
            pltpu.VMEM((2, CHUNK), jnp.int32),
            pltpu.VMEM((2, CHUNK), jnp.int32),
            pltpu.VMEM((2, CHUNK), jnp.int32),
            pltpu.VMEM((2, CHUNK, D), jnp.float32),
            pltpu.VMEM((ZB, D), jnp.float32),
            pltpu.VMEM((LANES,), jnp.int32),
            pltpu.VMEM_SHARED((acc_rows, D), jnp.float32),
            pltpu.SemaphoreType.DMA,
            pltpu.SemaphoreType.DMA,
            pltpu.SemaphoreType.DMA,
            pltpu.SemaphoreType.DMA,
            pltpu.SemaphoreType.DMA,
            pltpu.SemaphoreType.DMA,
        ],
    )
    def spmm_k(x_hbm, crow_hbm, ccol_hbm, cval_hbm, cnt_hbm, out_hbm,
               rows_v, cols_v, vals_v, gath_v, zero_v, cnt_v, acc,
               se0, se1, sg0, sg1, ss0, ss1):
        sem_e = (se0, se1)
        sem_g = (sg0, sg1)
        sem_s = (ss0, ss1)
        c = lax.axis_index("c")
        s = lax.axis_index("s")

        zvec = jnp.zeros((LANES,), jnp.float32)
        for r in range(ZB):
            for dd in range(D // LANES):
                zero_v[r, pl.ds(dd * LANES, LANES)] = zvec
        t0 = s * rows_per_tile

        @pl.loop(0, rows_per_tile, step=ZB)
        def _(off):
            pltpu.sync_copy(zero_v, acc.at[pl.ds(t0 + off, ZB)])

        pltpu.sync_copy(cnt_hbm.at[c, s], cnt_v)
        nck = jnp.max(cnt_v[pl.ds(0, LANES)], axis=0)

        plsc.subcore_barrier()

        ebase = s * cap

        def issue_idx(k, b):
            sl = pl.ds(pl.multiple_of(ebase + k * CHUNK, CHUNK), CHUNK)
            pltpu.async_copy(crow_hbm.at[c, sl], rows_v.at[b], sem_e[b])
            pltpu.async_copy(ccol_hbm.at[c, sl], cols_v.at[b], sem_e[b])
            pltpu.async_copy(cval_hbm.at[c, sl], vals_v.at[b], sem_e[b])

        def wait_idx(b):
            sl = pl.ds(0, CHUNK)
            pltpu.make_async_copy(crow_hbm.at[0, sl], rows_v.at[b],
                                  sem_e[b]).wait()
            pltpu.make_async_copy(ccol_hbm.at[0, sl], cols_v.at[b],
                                  sem_e[b]).wait()
            pltpu.make_async_copy(cval_hbm.at[0, sl], vals_v.at[b],
                                  sem_e[b]).wait()

        def issue_gather(b):
            pltpu.async_copy(x_hbm.at[cols_v.at[b]], gath_v.at[b], sem_g[b])

        def wait_gather(b):
            pltpu.make_async_copy(
                x_hbm.at[pl.ds(0, CHUNK)], gath_v.at[b], sem_g[b]).wait()

        def issue_scatter(b):
            pltpu.async_copy(gath_v.at[b], acc.at[rows_v.at[b]], sem_s[b],
                             add=True)

        def wait_scatter(b):
            pltpu.make_async_copy(
                gath_v.at[b], acc.at[pl.ds(0, CHUNK)], sem_s[b]).wait()

        def compute(b):
            for g in range(CHUNK // LANES):
                v16 = plsc.bitcast(
                    vals_v[b, pl.ds(g * LANES, LANES)], jnp.float32)
                for j in range(LANES):
                    e = g * LANES + j
                    ve = _splat(v16, j)
                    for dd in range(D // LANES):
                        sl = pl.ds(dd * LANES, LANES)
                        gath_v[b, e, sl] = gath_v[b, e, sl] * ve

        def body(k, b):
            wait_gather(b)
            compute(b)

            @pl.when(k + 1 < nck)
            def _():
                wait_idx(1 - b)

                @pl.when(k >= 1)
                def _():
                    wait_scatter(1 - b)

                issue_gather(1 - b)

            @pl.when(k + 2 < nck)
            def _():
                issue_idx(k + 2, b)

            issue_scatter(b)

        issue_idx(0, 0)
        wait_idx(0)
        issue_gather(0)
        issue_idx(1, 1)

        @pl.loop(0, nck, step=2)
        def _(k):
            body(k, 0)
            body(k + 1, 1)

        wait_scatter(1)
        plsc.subcore_barrier()

        out_base = c * acc_rows + s * rows_per_tile
        pltpu.sync_copy(acc.at[pl.ds(t0, rows_per_tile)],
                        out_hbm.at[pl.ds(out_base, rows_per_tile)])

    def partition(ebuf):
        return part_k(ebuf)

    def spmm(x, compact):
        crow, ccol, cval, cnt = compact
        out = spmm_k(x, crow, ccol, cval, cnt)
        return jnp.concatenate(
            [out[:half], out[acc_rows:acc_rows + half]], axis=0)

    return partition, spmm


_G_E_PAD = _ceil_to(800000, 2 * NTILES * CHUNK)
_II_E_PAD = _ceil_to(400000, 2 * NTILES * CHUNK)

_part_graph, _spmm_graph = _make_spmm(NU + NI, NU + NI, _G_E_PAD)
_part_ii, _spmm_ii = _make_spmm(NI, NI, _II_E_PAD)


def _pack_edges(indices, values, e_pad, n_rows):
    e = values.shape[0]
    pad = e_pad - e
    rows = jnp.concatenate([indices[0], jnp.full((pad,), n_rows, jnp.int32)])
    cols = jnp.concatenate([indices[1], jnp.zeros((pad,), jnp.int32)])
    vals = jnp.concatenate([values, jnp.zeros((pad,), jnp.float32)])
    nch = e_pad // CHUNK
    rec = jnp.concatenate([rows.reshape(nch, CHUNK),
                           cols.reshape(nch, CHUNK),
                           lax.bitcast_convert_type(vals, jnp.int32)
                              .reshape(nch, CHUNK)], axis=1)
    return rec


def kernel(user_emb, item_emb, graph_indices, graph_values, ii_indices, ii_values):
    ii_rec = _pack_edges(ii_indices, ii_values, _II_E_PAD, NI)
    ii_compact = _part_ii(ii_rec)
    cur = item_emb
    acc = item_emb
    for _ in range(2):
        cur = _spmm_ii(cur, ii_compact)
        acc = acc + cur
    items_emb = acc * (1.0 / 3.0)

    g_rec = _pack_edges(graph_indices, graph_values, _G_E_PAD, NU + NI)
    g_compact = _part_graph(g_rec)
    all_emb = jnp.concatenate([user_emb, items_emb], axis=0)
    s = all_emb
    for _ in range(3):
        all_emb = _spmm_graph(all_emb, g_compact)
        s = s + all_emb
    light = s * 0.25
    return light[:NU], light[NU:]


# speedup vs baseline: 7.7923x; 1.1497x over previous
import dataclasses
import functools

import jax
import jax.numpy as jnp
from jax import lax
from jax.experimental import pallas as pl
from jax.experimental.pallas import tpu as pltpu
from jax.experimental.pallas import tpu_sc as plsc

NU = 25000
NI = 25000
D = 64
LANES = 16
NTILES = 16
NSC = 2
CHUNK = 128
REC = 3 * CHUNK
ZB = 32
BLK = 1024
STG = 1536


def _ceil_to(x, m):
    return (x + m - 1) // m * m


_SPLAT_DNUMS = lax.GatherDimensionNumbers(
    offset_dims=(), collapsed_slice_dims=(0,), start_index_map=(0,))


def _splat(v16, j):
    idx = jnp.full((LANES, 1), j, jnp.int32)
    return lax.gather(v16, idx, _SPLAT_DNUMS, slice_sizes=(1,),
                      mode=lax.GatherScatterMode.PROMISE_IN_BOUNDS)


def _compiler_params():
    cp = pltpu.CompilerParams()
    if "needs_layout_passes" in pltpu.CompilerParams.__dataclass_fields__:
        cp = dataclasses.replace(cp, needs_layout_passes=False)
    if "use_tc_tiling_on_sc" in pltpu.CompilerParams.__dataclass_fields__:
        cp = dataclasses.replace(cp, use_tc_tiling_on_sc=False)
    return cp


_MESH = dict(core_axis_name="c", subcore_axis_name="s")


def _make_partition(n_out_rows, e_pad):
    half = n_out_rows // 2
    rows_per_tile = _ceil_to(-(-half // NTILES), ZB)
    acc_rows = rows_per_tile * NTILES
    dummy = half + LANES
    e_per_tile = e_pad // NTILES
    nc = e_per_tile // CHUNK
    cap = _ceil_to(nc * CHUNK, BLK) + BLK

    @functools.partial(
        pl.kernel,
        out_type=[
            jax.ShapeDtypeStruct((NSC, NTILES * cap), jnp.int32),
            jax.ShapeDtypeStruct((NSC, NTILES * cap), jnp.int32),
            jax.ShapeDtypeStruct((NSC, NTILES * cap), jnp.int32),
            jax.ShapeDtypeStruct((NSC, NTILES, LANES), jnp.int32),
        ],
        mesh=plsc.VectorSubcoreMesh(**_MESH),
        compiler_params=_compiler_params(),
        scratch_types=[
            pltpu.VMEM((2, REC), jnp.int32),
            pltpu.VMEM((STG,), jnp.int32),
            pltpu.VMEM((STG,), jnp.int32),
            pltpu.VMEM((STG,), jnp.int32),
            pltpu.VMEM((LANES,), jnp.int32),
            pltpu.SemaphoreType.DMA,
            pltpu.SemaphoreType.DMA,
        ],
    )
    def part_k(ebuf_hbm, orow_hbm, ocol_hbm, oval_hbm, ocnt_hbm,
               ebuf_v, srow_v, scol_v, sval_v, cnt_v, se0, se1):
        sem_e = (se0, se1)
        c = lax.axis_index("c")
        s = lax.axis_index("s")
        row_base = c * half
        c0 = s * nc
        obase = s * cap

        def issue_idx(k, b):
            pltpu.async_copy(ebuf_hbm.at[c0 + k], ebuf_v.at[b], sem_e[b])

        def wait_idx(b):
            pltpu.make_async_copy(
                ebuf_hbm.at[0], ebuf_v.at[b], sem_e[b]).wait()

        def flush(optr):
            off = pl.multiple_of(obase + optr, BLK)
            pltpu.sync_copy(srow_v.at[pl.ds(0, BLK)],
                            orow_hbm.at[c, pl.ds(off, BLK)])
            pltpu.sync_copy(scol_v.at[pl.ds(0, BLK)],
                            ocol_hbm.at[c, pl.ds(off, BLK)])
            pltpu.sync_copy(sval_v.at[pl.ds(0, BLK)],
                            oval_hbm.at[c, pl.ds(off, BLK)])

        def shift():
            for g in range(CHUNK // LANES):
                sl_src = pl.ds(BLK + g * LANES, LANES)
                sl_dst = pl.ds(g * LANES, LANES)
                srow_v[sl_dst] = srow_v[sl_src]
                scol_v[sl_dst] = scol_v[sl_src]
                sval_v[sl_dst] = sval_v[sl_src]

        issue_idx(0, 0)
        issue_idx(1, 1)

        def chunk_body(k, carry):
            cur, optr = carry
            b0 = k % 2
            def with_buf(b):
                wait_idx(b)

                @pl.when(k + 2 < nc)
                def _():
                    issue_idx(k + 2, b)

                cur2 = cur
                for g in range(CHUNK // LANES):
                    sl = pl.ds(g * LANES, LANES)
                    rows16 = ebuf_v[b, sl]
                    loc = rows16 - row_base
                    ok = (loc >= 0) & (loc < half)
                    n16 = plsc.all_reduce_population_count(ok)
                    cnt = jnp.max(n16, axis=0) if n16.ndim else n16
                    plsc.store_compressed(
                        srow_v.at[pl.ds(cur2, LANES)], loc, mask=ok)
                    plsc.store_compressed(
                        scol_v.at[pl.ds(cur2, LANES)],
                        ebuf_v[b, pl.ds(CHUNK + g * LANES, LANES)], mask=ok)
                    plsc.store_compressed(
                        sval_v.at[pl.ds(cur2, LANES)],
                        ebuf_v[b, pl.ds(2 * CHUNK + g * LANES, LANES)], mask=ok)
                    cur2 = cur2 + cnt
                return cur2

            cur = lax.cond(b0 == 0, lambda: with_buf(0), lambda: with_buf(1))

            def do_flush():
                flush(optr)
                shift()
                return cur - BLK, optr + BLK

            cur, optr = lax.cond(cur >= BLK, do_flush, lambda: (cur, optr))
            return cur, optr

        cur, optr = lax.fori_loop(0, nc, chunk_body, (jnp.int32(0),
                                                      jnp.int32(0)))

        zero16 = jnp.zeros((LANES,), jnp.int32)
        dummy16 = jnp.full((LANES,), dummy, jnp.int32)
        pad_to = _pad_target(cur)
        base = cur

        def pad_body(g, _):
            off = base + g * LANES

            @pl.when(off < pad_to)
            def _():
                srow_v[pl.ds(off, LANES)] = dummy16
                scol_v[pl.ds(off, LANES)] = zero16
                sval_v[pl.ds(off, LANES)] = zero16
            return 0

        lax.fori_loop(0, (STG - BLK) // LANES, pad_body, 0)
        cur = pad_to

        nrec = (optr + cur) // CHUNK

        def do_flush2():
            flush(optr)
            shift()
            return cur - BLK, optr + BLK

        cur, optr = lax.cond(cur >= BLK, do_flush2, lambda: (cur, optr))
        flush(optr)

        cnt_v[pl.ds(0, LANES)] = jnp.full((LANES,), 1, jnp.int32) * nrec
        pltpu.sync_copy(cnt_v, ocnt_hbm.at[c, s])

    return part_k, cap, acc_rows, rows_per_tile, dummy, half


def _pad_target(cur):
    rec = (cur + CHUNK - 1) // CHUNK
    rec = rec + (rec % 2)
    rec = jnp.maximum(rec, 2)
    return rec * CHUNK


def _make_spmm(n_x_rows, n_out_rows, e_pad):
    part_k, cap, acc_rows, rows_per_tile, dummy, half = _make_partition(
        n_out_rows, e_pad)

    @functools.partial(
        pl.kernel,
        out_type=jax.ShapeDtypeStruct((NSC * acc_rows, D), jnp.float32),
        mesh=plsc.VectorSubcoreMesh(**_MESH),
        compiler_params=_compiler_params(),
        scratch_types=[
            pltpu.VMEM((2, CHUNK), jnp.int32),
            pltpu.VMEM((2, CHUNK), jnp.int32),
            pltpu.VMEM((2, CHUNK), jnp.int32),
            pltpu.VMEM((2, CHUNK, D), jnp.float32),
            pltpu.VMEM((2, CHUNK), jnp.int32),
            pltpu.VMEM((ZB, D), jnp.float32),
            pltpu.VMEM((LANES,), jnp.int32),
            pltpu.VMEM_SHARED((acc_rows, D), jnp.float32),
            pltpu.SemaphoreType.DMA,
            pltpu.SemaphoreType.DMA,
            pltpu.SemaphoreType.DMA,
            pltpu.SemaphoreType.DMA,
            pltpu.SemaphoreType.DMA,
            pltpu.SemaphoreType.DMA,
        ],
    )
    def spmm_k(x_hbm, crow_hbm, ccol_hbm, cval_hbm, cnt_hbm, out_hbm,
               rows_v, cols_v, vals_v, gath_v, sidx_v, zero_v,
               cnt_v, acc, se0, se1, sg0, sg1, ss0, ss1):
        sem_e = (se0, se1)
        sem_g = (sg0, sg1)
        sem_s = (ss0, ss1)
        c = lax.axis_index("c")
        s = lax.axis_index("s")

        zvec = jnp.zeros((LANES,), jnp.float32)
        for r in range(ZB):
            for dd in range(D // LANES):
                zero_v[r, pl.ds(dd * LANES, LANES)] = zvec
        t0 = s * rows_per_tile

        @pl.loop(0, rows_per_tile, step=ZB)
        def _(off):
            pltpu.sync_copy(zero_v, acc.at[pl.ds(t0 + off, ZB)])

        pltpu.sync_copy(cnt_hbm.at[c, s], cnt_v)
        nck = jnp.max(cnt_v[pl.ds(0, LANES)], axis=0)

        plsc.subcore_barrier()

        ebase = s * cap

        def issue_idx(k, b):
            sl = pl.ds(pl.multiple_of(ebase + k * CHUNK, CHUNK), CHUNK)
            pltpu.async_copy(crow_hbm.at[c, sl], rows_v.at[b], sem_e[b])
            pltpu.async_copy(ccol_hbm.at[c, sl], cols_v.at[b], sem_e[b])
            pltpu.async_copy(cval_hbm.at[c, sl], vals_v.at[b], sem_e[b])

        def wait_idx(b):
            sl = pl.ds(0, CHUNK)
            pltpu.make_async_copy(crow_hbm.at[0, sl], rows_v.at[b],
                                  sem_e[b]).wait()
            pltpu.make_async_copy(ccol_hbm.at[0, sl], cols_v.at[b],
                                  sem_e[b]).wait()
            pltpu.make_async_copy(cval_hbm.at[0, sl], vals_v.at[b],
                                  sem_e[b]).wait()

        def issue_gather(b):
            pltpu.async_copy(x_hbm.at[cols_v.at[b]], gath_v.at[b], sem_g[b])

        def wait_gather(b):
            pltpu.make_async_copy(
                x_hbm.at[pl.ds(0, CHUNK)], gath_v.at[b], sem_g[b]).wait()

        def issue_scatter(b):
            pltpu.async_copy(gath_v.at[b], acc.at[sidx_v.at[b]], sem_s[b],
                             add=True)

        def wait_scatter(b):
            pltpu.make_async_copy(
                gath_v.at[b], acc.at[pl.ds(0, CHUNK)], sem_s[b]).wait()

        def compute(b):
            for g in range(CHUNK // LANES):
                sl = pl.ds(g * LANES, LANES)
                sidx_v[b, sl] = rows_v[b, sl]
                v16 = plsc.bitcast(vals_v[b, sl], jnp.float32)
                for j in range(LANES):
                    e = g * LANES + j
                    ve = _splat(v16, j)
                    for dd in range(D // LANES):
                        sld = pl.ds(dd * LANES, LANES)
                        gath_v[b, e, sld] = gath_v[b, e, sld] * ve

        def body(k, b):
            @pl.when(k + 1 < nck)
            def _():
                wait_idx(1 - b)

                @pl.when(k >= 1)
                def _():
                    wait_scatter(1 - b)

                issue_gather(1 - b)

            wait_gather(b)
            compute(b)

            @pl.when(k + 2 < nck)
            def _():
                issue_idx(k + 2, b)

            issue_scatter(b)

        issue_idx(0, 0)
        wait_idx(0)
        issue_gather(0)
        issue_idx(1, 1)

        @pl.loop(0, nck, step=2)
        def _(k):
            body(k, 0)
            body(k + 1, 1)

        wait_scatter(1)
        plsc.subcore_barrier()

        out_base = c * acc_rows + s * rows_per_tile
        pltpu.sync_copy(acc.at[pl.ds(t0, rows_per_tile)],
                        out_hbm.at[pl.ds(out_base, rows_per_tile)])

    def partition(ebuf):
        return part_k(ebuf)

    def spmm(x, compact):
        crow, ccol, cval, cnt = compact
        out = spmm_k(x, crow, ccol, cval, cnt)
        return jnp.concatenate(
            [out[:half], out[acc_rows:acc_rows + half]], axis=0)

    return partition, spmm


_G_E_PAD = _ceil_to(800000, 2 * NTILES * CHUNK)
_II_E_PAD = _ceil_to(400000, 2 * NTILES * CHUNK)

_part_graph, _spmm_graph = _make_spmm(NU + NI, NU + NI, _G_E_PAD)
_part_ii, _spmm_ii = _make_spmm(NI, NI, _II_E_PAD)


def _pack_edges(indices, values, e_pad, n_rows):
    e = values.shape[0]
    pad = e_pad - e
    rows = jnp.concatenate([indices[0], jnp.full((pad,), n_rows, jnp.int32)])
    cols = jnp.concatenate([indices[1], jnp.zeros((pad,), jnp.int32)])
    vals = jnp.concatenate([values, jnp.zeros((pad,), jnp.float32)])
    nch = e_pad // CHUNK
    rec = jnp.concatenate([rows.reshape(nch, CHUNK),
                           cols.reshape(nch, CHUNK),
                           lax.bitcast_convert_type(vals, jnp.int32)
                              .reshape(nch, CHUNK)], axis=1)
    return rec


def kernel(user_emb, item_emb, graph_indices, graph_values, ii_indices, ii_values):
    ii_rec = _pack_edges(ii_indices, ii_values, _II_E_PAD, NI)
    ii_compact = _part_ii(ii_rec)
    cur = item_emb
    acc = item_emb
    for _ in range(2):
        cur = _spmm_ii(cur, ii_compact)
        acc = acc + cur
    items_emb = acc * (1.0 / 3.0)

    g_rec = _pack_edges(graph_indices, graph_values, _G_E_PAD, NU + NI)
    g_compact = _part_graph(g_rec)
    all_emb = jnp.concatenate([user_emb, items_emb], axis=0)
    s = all_emb
    for _ in range(3):
        all_emb = _spmm_graph(all_emb, g_compact)
        s = s + all_emb
    light = s * 0.25
    return light[:NU], light[NU:]


# speedup vs baseline: 7.8050x; 1.0016x over previous
import dataclasses
import functools

import jax
import jax.numpy as jnp
from jax import lax
from jax.experimental import pallas as pl
from jax.experimental.pallas import tpu as pltpu
from jax.experimental.pallas import tpu_sc as plsc

NU = 25000
NI = 25000
D = 64
LANES = 16
NTILES = 16
NSC = 2
CHUNK = 128
REC = 3 * CHUNK
ZB = 32
BLK = 1024
STG = 1536


def _ceil_to(x, m):
    return (x + m - 1) // m * m


_SPLAT_DNUMS = lax.GatherDimensionNumbers(
    offset_dims=(), collapsed_slice_dims=(0,), start_index_map=(0,))


def _splat(v16, j):
    idx = jnp.full((LANES, 1), j, jnp.int32)
    return lax.gather(v16, idx, _SPLAT_DNUMS, slice_sizes=(1,),
                      mode=lax.GatherScatterMode.PROMISE_IN_BOUNDS)


def _compiler_params():
    cp = pltpu.CompilerParams()
    if "needs_layout_passes" in pltpu.CompilerParams.__dataclass_fields__:
        cp = dataclasses.replace(cp, needs_layout_passes=False)
    if "use_tc_tiling_on_sc" in pltpu.CompilerParams.__dataclass_fields__:
        cp = dataclasses.replace(cp, use_tc_tiling_on_sc=False)
    return cp


_MESH = dict(core_axis_name="c", subcore_axis_name="s")


def _make_partition(n_out_rows, e_pad):
    half = n_out_rows // 2
    rows_per_tile = _ceil_to(-(-half // NTILES), ZB)
    acc_rows = rows_per_tile * NTILES
    dummy = half + LANES
    e_per_tile = e_pad // NTILES
    nc = e_per_tile // CHUNK
    cap = _ceil_to(nc * CHUNK, BLK) + BLK

    @functools.partial(
        pl.kernel,
        out_type=[
            jax.ShapeDtypeStruct((NSC, NTILES * cap), jnp.int32),
            jax.ShapeDtypeStruct((NSC, NTILES * cap), jnp.int32),
            jax.ShapeDtypeStruct((NSC, NTILES * cap), jnp.int32),
            jax.ShapeDtypeStruct((NSC, NTILES, LANES), jnp.int32),
        ],
        mesh=plsc.VectorSubcoreMesh(**_MESH),
        compiler_params=_compiler_params(),
        scratch_types=[
            pltpu.VMEM((2, REC), jnp.int32),
            pltpu.VMEM((STG,), jnp.int32),
            pltpu.VMEM((STG,), jnp.int32),
            pltpu.VMEM((STG,), jnp.int32),
            pltpu.VMEM((LANES,), jnp.int32),
            pltpu.SemaphoreType.DMA,
            pltpu.SemaphoreType.DMA,
        ],
    )
    def part_k(ebuf_hbm, orow_hbm, ocol_hbm, oval_hbm, ocnt_hbm,
               ebuf_v, srow_v, scol_v, sval_v, cnt_v, se0, se1):
        sem_e = (se0, se1)
        c = lax.axis_index("c")
        s = lax.axis_index("s")
        row_base = c * half
        c0 = s * nc
        obase = s * cap

        def issue_idx(k, b):
            pltpu.async_copy(ebuf_hbm.at[c0 + k], ebuf_v.at[b], sem_e[b])

        def wait_idx(b):
            pltpu.make_async_copy(
                ebuf_hbm.at[0], ebuf_v.at[b], sem_e[b]).wait()

        def flush(optr):
            off = pl.multiple_of(obase + optr, BLK)
            pltpu.sync_copy(srow_v.at[pl.ds(0, BLK)],
                            orow_hbm.at[c, pl.ds(off, BLK)])
            pltpu.sync_copy(scol_v.at[pl.ds(0, BLK)],
                            ocol_hbm.at[c, pl.ds(off, BLK)])
            pltpu.sync_copy(sval_v.at[pl.ds(0, BLK)],
                            oval_hbm.at[c, pl.ds(off, BLK)])

        def shift():
            for g in range(CHUNK // LANES):
                sl_src = pl.ds(BLK + g * LANES, LANES)
                sl_dst = pl.ds(g * LANES, LANES)
                srow_v[sl_dst] = srow_v[sl_src]
                scol_v[sl_dst] = scol_v[sl_src]
                sval_v[sl_dst] = sval_v[sl_src]

        issue_idx(0, 0)
        issue_idx(1, 1)

        def chunk_body(k, carry):
            cur, optr = carry
            b0 = k % 2
            def with_buf(b):
                wait_idx(b)

                @pl.when(k + 2 < nc)
                def _():
                    issue_idx(k + 2, b)

                cur2 = cur
                for g in range(CHUNK // LANES):
                    sl = pl.ds(g * LANES, LANES)
                    rows16 = ebuf_v[b, sl]
                    loc = rows16 - row_base
                    ok = (loc >= 0) & (loc < half)
                    n16 = plsc.all_reduce_population_count(ok)
                    cnt = jnp.max(n16, axis=0) if n16.ndim else n16
                    plsc.store_compressed(
                        srow_v.at[pl.ds(cur2, LANES)], loc, mask=ok)
                    plsc.store_compressed(
                        scol_v.at[pl.ds(cur2, LANES)],
                        ebuf_v[b, pl.ds(CHUNK + g * LANES, LANES)], mask=ok)
                    plsc.store_compressed(
                        sval_v.at[pl.ds(cur2, LANES)],
                        ebuf_v[b, pl.ds(2 * CHUNK + g * LANES, LANES)], mask=ok)
                    cur2 = cur2 + cnt
                return cur2

            cur = lax.cond(b0 == 0, lambda: with_buf(0), lambda: with_buf(1))

            def do_flush():
                flush(optr)
                shift()
                return cur - BLK, optr + BLK

            cur, optr = lax.cond(cur >= BLK, do_flush, lambda: (cur, optr))
            return cur, optr

        cur, optr = lax.fori_loop(0, nc, chunk_body, (jnp.int32(0),
                                                      jnp.int32(0)))

        zero16 = jnp.zeros((LANES,), jnp.int32)
        dummy16 = jnp.full((LANES,), dummy, jnp.int32)
        pad_to = _pad_target(cur)
        base = cur

        def pad_body(g, _):
            off = base + g * LANES

            @pl.when(off < pad_to)
            def _():
                srow_v[pl.ds(off, LANES)] = dummy16
                scol_v[pl.ds(off, LANES)] = zero16
                sval_v[pl.ds(off, LANES)] = zero16
            return 0

        lax.fori_loop(0, (STG - BLK) // LANES, pad_body, 0)
        cur = pad_to

        nrec = (optr + cur) // CHUNK

        def do_flush2():
            flush(optr)
            shift()
            return cur - BLK, optr + BLK

        cur, optr = lax.cond(cur >= BLK, do_flush2, lambda: (cur, optr))
        flush(optr)

        cnt_v[pl.ds(0, LANES)] = jnp.full((LANES,), 1, jnp.int32) * nrec
        pltpu.sync_copy(cnt_v, ocnt_hbm.at[c, s])

    return part_k, cap, acc_rows, rows_per_tile, dummy, half


def _pad_target(cur):
    rec = (cur + CHUNK - 1) // CHUNK
    rec = rec + (rec % 2)
    rec = jnp.maximum(rec, 2)
    return rec * CHUNK


def _make_spmm(n_x_rows, n_out_rows, e_pad):
    part_k, cap, acc_rows, rows_per_tile, dummy, half = _make_partition(
        n_out_rows, e_pad)

    @functools.partial(
        pl.kernel,
        out_type=jax.ShapeDtypeStruct((NSC * acc_rows, D), jnp.float32),
        mesh=plsc.VectorSubcoreMesh(**_MESH),
        compiler_params=_compiler_params(),
        scratch_types=[
            pltpu.VMEM((2, CHUNK), jnp.int32),
            pltpu.VMEM((2, CHUNK), jnp.int32),
            pltpu.VMEM((2, CHUNK), jnp.int32),
            pltpu.VMEM((2, CHUNK, D), jnp.float32),
            pltpu.VMEM((2, CHUNK), jnp.int32),
            pltpu.VMEM((ZB, D), jnp.float32),
            pltpu.VMEM((LANES,), jnp.int32),
            pltpu.VMEM_SHARED((acc_rows, D), jnp.float32),
            pltpu.SemaphoreType.DMA,
            pltpu.SemaphoreType.DMA,
            pltpu.SemaphoreType.DMA,
            pltpu.SemaphoreType.DMA,
            pltpu.SemaphoreType.DMA,
            pltpu.SemaphoreType.DMA,
        ],
    )
    def spmm_k(x_hbm, crow_hbm, ccol_hbm, cval_hbm, cnt_hbm, out_hbm,
               rows_v, cols_v, vals_v, gath_v, sidx_v, zero_v,
               cnt_v, acc, se0, se1, sg0, sg1, ss0, ss1):
        sem_e = (se0, se1)
        sem_g = (sg0, sg1)
        sem_s = (ss0, ss1)
        c = lax.axis_index("c")
        s = lax.axis_index("s")

        zvec = jnp.zeros((LANES,), jnp.float32)
        for r in range(ZB):
            for dd in range(D // LANES):
                zero_v[r, pl.ds(dd * LANES, LANES)] = zvec
        t0 = s * rows_per_tile

        pltpu.sync_copy(cnt_hbm.at[c, s], cnt_v)
        nck = jnp.max(cnt_v[pl.ds(0, LANES)], axis=0)

        ebase = s * cap

        def issue_idx(k, b):
            sl = pl.ds(pl.multiple_of(ebase + k * CHUNK, CHUNK), CHUNK)
            pltpu.async_copy(crow_hbm.at[c, sl], rows_v.at[b], sem_e[b])
            pltpu.async_copy(ccol_hbm.at[c, sl], cols_v.at[b], sem_e[b])
            pltpu.async_copy(cval_hbm.at[c, sl], vals_v.at[b], sem_e[b])

        def wait_idx(b):
            sl = pl.ds(0, CHUNK)
            pltpu.make_async_copy(crow_hbm.at[0, sl], rows_v.at[b],
                                  sem_e[b]).wait()
            pltpu.make_async_copy(ccol_hbm.at[0, sl], cols_v.at[b],
                                  sem_e[b]).wait()
            pltpu.make_async_copy(cval_hbm.at[0, sl], vals_v.at[b],
                                  sem_e[b]).wait()

        def issue_gather(b):
            pltpu.async_copy(x_hbm.at[cols_v.at[b]], gath_v.at[b], sem_g[b])

        def wait_gather(b):
            pltpu.make_async_copy(
                x_hbm.at[pl.ds(0, CHUNK)], gath_v.at[b], sem_g[b]).wait()

        def issue_scatter(b):
            pltpu.async_copy(gath_v.at[b], acc.at[sidx_v.at[b]], sem_s[b],
                             add=True)

        def wait_scatter(b):
            pltpu.make_async_copy(
                gath_v.at[b], acc.at[pl.ds(0, CHUNK)], sem_s[b]).wait()

        def compute(b):
            for g in range(CHUNK // LANES):
                sl = pl.ds(g * LANES, LANES)
                sidx_v[b, sl] = rows_v[b, sl]
                v16 = plsc.bitcast(vals_v[b, sl], jnp.float32)
                for j in range(LANES):
                    e = g * LANES + j
                    ve = _splat(v16, j)
                    for dd in range(D // LANES):
                        sld = pl.ds(dd * LANES, LANES)
                        gath_v[b, e, sld] = gath_v[b, e, sld] * ve

        def body(k, b):
            @pl.when(k + 1 < nck)
            def _():
                wait_idx(1 - b)

                @pl.when(k >= 1)
                def _():
                    wait_scatter(1 - b)

                issue_gather(1 - b)

            wait_gather(b)
            compute(b)

            @pl.when(k + 2 < nck)
            def _():
                issue_idx(k + 2, b)

            issue_scatter(b)

        issue_idx(0, 0)
        issue_idx(1, 1)

        wait_idx(0)
        issue_gather(0)

        @pl.loop(0, rows_per_tile, step=ZB)
        def _(off):
            pltpu.sync_copy(zero_v, acc.at[pl.ds(t0 + off, ZB)])

        plsc.subcore_barrier()

        @pl.loop(0, nck, step=2)
        def _(k):
            body(k, 0)
            body(k + 1, 1)

        wait_scatter(1)
        plsc.subcore_barrier()

        out_base = c * acc_rows + s * rows_per_tile
        pltpu.sync_copy(acc.at[pl.ds(t0, rows_per_tile)],
                        out_hbm.at[pl.ds(out_base, rows_per_tile)])

    def partition(ebuf):
        return part_k(ebuf)

    def spmm(x, compact):
        crow, ccol, cval, cnt = compact
        out = spmm_k(x, crow, ccol, cval, cnt)
        return jnp.concatenate(
            [out[:half], out[acc_rows:acc_rows + half]], axis=0)

    return partition, spmm


_G_E_PAD = _ceil_to(800000, 2 * NTILES * CHUNK)
_II_E_PAD = _ceil_to(400000, 2 * NTILES * CHUNK)

_part_graph, _spmm_graph = _make_spmm(NU + NI, NU + NI, _G_E_PAD)
_part_ii, _spmm_ii = _make_spmm(NI, NI, _II_E_PAD)


def _pack_edges(indices, values, e_pad, n_rows):
    e = values.shape[0]
    pad = e_pad - e
    rows = jnp.concatenate([indices[0], jnp.full((pad,), n_rows, jnp.int32)])
    cols = jnp.concatenate([indices[1], jnp.zeros((pad,), jnp.int32)])
    vals = jnp.concatenate([values, jnp.zeros((pad,), jnp.float32)])
    nch = e_pad // CHUNK
    rec = jnp.concatenate([rows.reshape(nch, CHUNK),
                           cols.reshape(nch, CHUNK),
                           lax.bitcast_convert_type(vals, jnp.int32)
                              .reshape(nch, CHUNK)], axis=1)
    return rec


def kernel(user_emb, item_emb, graph_indices, graph_values, ii_indices, ii_values):
    ii_rec = _pack_edges(ii_indices, ii_values, _II_E_PAD, NI)
    ii_compact = _part_ii(ii_rec)
    cur = item_emb
    acc = item_emb
    for _ in range(2):
        cur = _spmm_ii(cur, ii_compact)
        acc = acc + cur
    items_emb = acc * (1.0 / 3.0)

    g_rec = _pack_edges(graph_indices, graph_values, _G_E_PAD, NU + NI)
    g_compact = _part_graph(g_rec)
    all_emb = jnp.concatenate([user_emb, items_emb], axis=0)
    s = all_emb
    for _ in range(3):
        all_emb = _spmm_graph(all_emb, g_compact)
        s = s + all_emb
    light = s * 0.25
    return light[:NU], light[NU:]
